# Initial kernel scaffold; baseline (speedup 1.0000x reference)
import jax
import jax.numpy as jnp
from jax.experimental import pallas as pl


def kernel(z_t, g_t, prototypes):
    raise NotImplementedError("write your pallas kernel here")


# speedup vs baseline: 18.3106x; 18.3106x over previous
import functools

import jax
import jax.numpy as jnp
from jax.experimental import pallas as pl
from jax.experimental.pallas import tpu as pltpu

MU = 0.9
OMU = 1.0 - MU
DELTA = 1.0
ALPHA_P = 1.0
ALPHA_R = 0.5
TOPK = 3

N = 2048
DIM = 2048
K = 1024

B = 64
NB = N // B
BF = 256
NBF = N // BF

_HI = jax.lax.Precision.HIGHEST


def _dot_t(a, b):
    return jax.lax.dot_general(
        a, b, (((1,), (1,)), ((), ())),
        preferred_element_type=jnp.float32, precision=_HI)


def _sum11(x):
    return jnp.sum(jnp.sum(x, axis=1, keepdims=True), axis=0, keepdims=True)


def _scan_kernel(zb_ref, gb_ref, p0_ref,
                 p_ref, n2_ref, za_ref, cnt_ref,
                 d_s, gb_s):
    i = pl.program_id(0)

    @pl.when(i == 0)
    def _init():
        p0 = p0_ref[...]
        p_ref[...] = p0
        n2col = jnp.sum(p0 * p0, axis=1, keepdims=True)
        n2_ref[...] = n2col.T
        za_ref[...] = jnp.zeros_like(za_ref)
        cnt_ref[...] = jnp.zeros_like(cnt_ref)

    zb = zb_ref[...]
    gb = gb_ref[...]

    d_s[...] = _dot_t(zb, p_ref[...])
    gb_s[...] = _dot_t(zb, zb)

    za_ref[...] += jnp.sum((1.0 - gb) * zb, axis=0, keepdims=True)
    cnt_ref[...] += _sum11(1.0 - gb)

    iota_k = jax.lax.broadcasted_iota(jnp.int32, (1, K), 1)
    iota_b = jax.lax.broadcasted_iota(jnp.int32, (1, B), 1)

    def body(r, carry):
        n2v = n2_ref[...]
        drow = d_s[pl.ds(r, 1), :]
        d2row = n2v - 2.0 * drow
        mval = jnp.min(d2row)
        k = jnp.min(jnp.where(d2row == mval, iota_k, K))
        gr = gb_ref[r, 0]

        oh = (iota_k == k).astype(jnp.float32)
        ohg = oh * gr
        ohb = (iota_b == r).astype(jnp.float32)

        gball = gb_s[...]
        grr = jnp.sum(gb_s[pl.ds(r, 1), :] * ohb)
        gcol = jnp.sum(gball * ohb, axis=1, keepdims=True)
        dkr = jnp.sum(drow * oh)
        n2k = jnp.sum(n2v * oh)

        n2p = (MU * MU) * n2k + (2.0 * MU * OMU) * dkr + (OMU * OMU) * grr
        n2_ref[...] = n2v + ohg * (n2p - n2k)

        dall = d_s[...]
        d_s[...] = dall + ohg * (OMU * (gcol - dall))

        prow = p_ref[pl.ds(k, 1), :]
        zrow = zb_ref[pl.ds(r, 1), :]
        scale = MU * gr + (1.0 - gr)
        p_ref[pl.ds(k, 1), :] = scale * prow + (OMU * gr) * zrow
        return carry

    jax.lax.fori_loop(0, B, body, 0, unroll=False)


def _loss_kernel(zb_ref, p_ref, n2_ref, za_ref, cnt_ref, out_ref):
    i = pl.program_id(0)

    @pl.when(i == 0)
    def _init():
        out_ref[...] = jnp.zeros_like(out_ref)

    zb = zb_ref[...]
    p = p_ref[...]
    n2 = n2_ref[...]
    z2 = jnp.sum(zb * zb, axis=1, keepdims=True)

    d2 = z2 + n2 - 2.0 * _dot_t(zb, p)

    iota_k = jax.lax.broadcasted_iota(jnp.int32, (1, K), 1)
    work = d2
    asum = jnp.zeros_like(d2)
    for _ in range(TOPK):
        m = jnp.min(work, axis=1, keepdims=True)
        first = jnp.min(jnp.where(work == m, iota_k, K), axis=1, keepdims=True)
        sel = (iota_k == first).astype(jnp.float32)
        asum = asum + sel
        work = jnp.where(sel > 0.0, jnp.float32(jnp.inf), work)

    mbar = jax.lax.dot_general(
        asum, p, (((1,), (0,)), ((), ())),
        preferred_element_type=jnp.float32, precision=_HI) / 3.0
    diff = zb - mbar
    lpull = _sum11(diff * diff)

    cnt = cnt_ref[...]
    ma = za_ref[...] / jnp.maximum(cnt, 1.0)
    ma2 = jnp.sum(ma * ma, axis=1, keepdims=True)
    zma = _dot_t(zb, ma)
    dist = jnp.sqrt(jnp.maximum(z2 - 2.0 * zma + ma2, 0.0))
    hinge = jnp.maximum(DELTA - dist, 0.0)
    lpush = _sum11(hinge)

    gate = (cnt > 0.0).astype(jnp.float32)
    out_ref[...] += (ALPHA_P / (N * DIM)) * lpull \
        + (ALPHA_R / N) * gate * lpush


@jax.jit
def kernel(z_t, g_t, prototypes):
    g2 = g_t.reshape(N, 1).astype(jnp.float32)

    p_fin, n2_fin, za, cnt = pl.pallas_call(
        _scan_kernel,
        grid=(NB,),
        in_specs=[
            pl.BlockSpec((B, DIM), lambda i: (i, 0)),
            pl.BlockSpec((B, 1), lambda i: (i, 0)),
            pl.BlockSpec((K, DIM), lambda i: (0, 0)),
        ],
        out_specs=[
            pl.BlockSpec((K, DIM), lambda i: (0, 0)),
            pl.BlockSpec((1, K), lambda i: (0, 0)),
            pl.BlockSpec((1, DIM), lambda i: (0, 0)),
            pl.BlockSpec((1, 1), lambda i: (0, 0)),
        ],
        out_shape=[
            jax.ShapeDtypeStruct((K, DIM), jnp.float32),
            jax.ShapeDtypeStruct((1, K), jnp.float32),
            jax.ShapeDtypeStruct((1, DIM), jnp.float32),
            jax.ShapeDtypeStruct((1, 1), jnp.float32),
        ],
        scratch_shapes=[
            pltpu.VMEM((B, K), jnp.float32),
            pltpu.VMEM((B, B), jnp.float32),
        ],
    )(z_t, g2, prototypes)

    out = pl.pallas_call(
        _loss_kernel,
        grid=(NBF,),
        in_specs=[
            pl.BlockSpec((BF, DIM), lambda i: (i, 0)),
            pl.BlockSpec((K, DIM), lambda i: (0, 0)),
            pl.BlockSpec((1, K), lambda i: (0, 0)),
            pl.BlockSpec((1, DIM), lambda i: (0, 0)),
            pl.BlockSpec((1, 1), lambda i: (0, 0)),
        ],
        out_specs=pl.BlockSpec((1, 1), lambda i: (0, 0)),
        out_shape=jax.ShapeDtypeStruct((1, 1), jnp.float32),
    )(z_t, p_fin, n2_fin, za, cnt)

    return out.reshape(())


# speedup vs baseline: 22.3534x; 1.2208x over previous
import functools

import jax
import jax.numpy as jnp
from jax.experimental import pallas as pl
from jax.experimental.pallas import tpu as pltpu

MU = 0.9
OMU = 1.0 - MU
DELTA = 1.0
ALPHA_P = 1.0
ALPHA_R = 0.5
TOPK = 3

N = 2048
DIM = 2048
K = 1024

B = 64
NB = N // B
BF = 256
NBF = N // BF

_HI = jax.lax.Precision.HIGHEST


def _dot_t(a, b):
    return jax.lax.dot_general(
        a, b, (((1,), (1,)), ((), ())),
        preferred_element_type=jnp.float32, precision=_HI)


def _sum11(x):
    return jnp.sum(jnp.sum(x, axis=1, keepdims=True), axis=0, keepdims=True)


def _scan_kernel(zb_ref, gb_ref, p0_ref,
                 p_ref, n2_ref, za_ref, cnt_ref,
                 d_s, gb_s):
    i = pl.program_id(0)

    @pl.when(i == 0)
    def _init():
        p0 = p0_ref[...]
        p_ref[...] = p0
        n2col = jnp.sum(p0 * p0, axis=1, keepdims=True)
        n2_ref[...] = n2col.T
        za_ref[...] = jnp.zeros_like(za_ref)
        cnt_ref[...] = jnp.zeros_like(cnt_ref)

    zb = zb_ref[...]
    gb = gb_ref[...]

    d_s[...] = _dot_t(zb, p_ref[...])
    gb_s[...] = _dot_t(zb, zb)

    za_ref[...] += jnp.sum((1.0 - gb) * zb, axis=0, keepdims=True)
    cnt_ref[...] += _sum11(1.0 - gb)

    iota_k = jax.lax.broadcasted_iota(jnp.int32, (1, K), 1)
    iota_b = jax.lax.broadcasted_iota(jnp.int32, (1, B), 1)

    def body(r, carry):
        n2v = n2_ref[...]
        drow = d_s[pl.ds(r, 1), :]
        d2row = n2v - 2.0 * drow
        mval = jnp.min(d2row)
        k = jnp.min(jnp.where(d2row == mval, iota_k, K))
        gr = gb_ref[r, 0]

        oh = (iota_k == k).astype(jnp.float32)
        ohg = oh * gr
        ohb = (iota_b == r).astype(jnp.float32)

        gball = gb_s[...]
        grr = jnp.sum(gb_s[pl.ds(r, 1), :] * ohb)
        gcol = jnp.sum(gball * ohb, axis=1, keepdims=True)

        n2_ref[...] = n2v + ohg * (
            (MU * MU - 1.0) * n2v + (2.0 * MU * OMU) * drow
            + (OMU * OMU) * grr)

        dall = d_s[...]
        d_s[...] = dall + ohg * (OMU * (gcol - dall))

        prow = p_ref[pl.ds(k, 1), :]
        zrow = zb_ref[pl.ds(r, 1), :]
        scale = MU * gr + (1.0 - gr)
        p_ref[pl.ds(k, 1), :] = scale * prow + (OMU * gr) * zrow
        return carry

    jax.lax.fori_loop(0, B, body, 0, unroll=False)


def _loss_kernel(zb_ref, p_ref, n2_ref, za_ref, cnt_ref, out_ref):
    i = pl.program_id(0)

    @pl.when(i == 0)
    def _init():
        out_ref[...] = jnp.zeros_like(out_ref)

    zb = zb_ref[...]
    p = p_ref[...]
    n2 = n2_ref[...]
    z2 = jnp.sum(zb * zb, axis=1, keepdims=True)

    d2 = z2 + n2 - 2.0 * _dot_t(zb, p)

    iota_k = jax.lax.broadcasted_iota(jnp.int32, (1, K), 1)
    work = d2
    asum = jnp.zeros_like(d2)
    for _ in range(TOPK):
        m = jnp.min(work, axis=1, keepdims=True)
        first = jnp.min(jnp.where(work == m, iota_k, K), axis=1, keepdims=True)
        sel = (iota_k == first).astype(jnp.float32)
        asum = asum + sel
        work = jnp.where(sel > 0.0, jnp.float32(jnp.inf), work)

    mbar = jax.lax.dot_general(
        asum, p, (((1,), (0,)), ((), ())),
        preferred_element_type=jnp.float32, precision=_HI) / 3.0
    diff = zb - mbar
    lpull = _sum11(diff * diff)

    cnt = cnt_ref[...]
    ma = za_ref[...] / jnp.maximum(cnt, 1.0)
    ma2 = jnp.sum(ma * ma, axis=1, keepdims=True)
    zma = _dot_t(zb, ma)
    dist = jnp.sqrt(jnp.maximum(z2 - 2.0 * zma + ma2, 0.0))
    hinge = jnp.maximum(DELTA - dist, 0.0)
    lpush = _sum11(hinge)

    gate = (cnt > 0.0).astype(jnp.float32)
    out_ref[...] += (ALPHA_P / (N * DIM)) * lpull \
        + (ALPHA_R / N) * gate * lpush


@jax.jit
def kernel(z_t, g_t, prototypes):
    g2 = g_t.reshape(N, 1).astype(jnp.float32)

    p_fin, n2_fin, za, cnt = pl.pallas_call(
        _scan_kernel,
        grid=(NB,),
        in_specs=[
            pl.BlockSpec((B, DIM), lambda i: (i, 0)),
            pl.BlockSpec((B, 1), lambda i: (i, 0)),
            pl.BlockSpec((K, DIM), lambda i: (0, 0)),
        ],
        out_specs=[
            pl.BlockSpec((K, DIM), lambda i: (0, 0)),
            pl.BlockSpec((1, K), lambda i: (0, 0)),
            pl.BlockSpec((1, DIM), lambda i: (0, 0)),
            pl.BlockSpec((1, 1), lambda i: (0, 0)),
        ],
        out_shape=[
            jax.ShapeDtypeStruct((K, DIM), jnp.float32),
            jax.ShapeDtypeStruct((1, K), jnp.float32),
            jax.ShapeDtypeStruct((1, DIM), jnp.float32),
            jax.ShapeDtypeStruct((1, 1), jnp.float32),
        ],
        scratch_shapes=[
            pltpu.VMEM((B, K), jnp.float32),
            pltpu.VMEM((B, B), jnp.float32),
        ],
    )(z_t, g2, prototypes)

    out = pl.pallas_call(
        _loss_kernel,
        grid=(NBF,),
        in_specs=[
            pl.BlockSpec((BF, DIM), lambda i: (i, 0)),
            pl.BlockSpec((K, DIM), lambda i: (0, 0)),
            pl.BlockSpec((1, K), lambda i: (0, 0)),
            pl.BlockSpec((1, DIM), lambda i: (0, 0)),
            pl.BlockSpec((1, 1), lambda i: (0, 0)),
        ],
        out_specs=pl.BlockSpec((1, 1), lambda i: (0, 0)),
        out_shape=jax.ShapeDtypeStruct((1, 1), jnp.float32),
    )(z_t, p_fin, n2_fin, za, cnt)

    return out.reshape(())
